# baseline (speedup 1.0000x reference)
import functools

import jax
import jax.numpy as jnp
from jax import lax
from jax.experimental import pallas as pl
from jax.experimental.pallas import tpu as pltpu
from jax.experimental.pallas import tpu_sc as plsc

_NC = 2
_NS = 16
_CB = 112
_NBUF = 4


def _make_sc_degree(n_pad, chunks_per_tile):
    sl = n_pad // _NS

    mesh = plsc.VectorSubcoreMesh(core_axis_name="c", subcore_axis_name="s")

    @functools.partial(
        pl.kernel,
        out_type=jax.ShapeDtypeStruct((_NC, n_pad), jnp.float32),
        mesh=mesh,
        scratch_types=[
            pltpu.VMEM((chunks_per_tile, _CB), jnp.int32),
            pltpu.VMEM((128,), jnp.float32),
            pltpu.VMEM_SHARED((n_pad,), jnp.float32),
        ],
    )
    def deg_kernel(dst3, ones_hbm, zeros_hbm, out, idx_v, ones_v, deg_sh):
        c = lax.axis_index("c")
        s = lax.axis_index("s")
        wid = c * _NS + s
        pltpu.sync_copy(dst3.at[wid], idx_v)
        pltpu.sync_copy(ones_hbm, ones_v)
        pltpu.sync_copy(zeros_hbm.at[pl.ds(s * sl, sl)],
                        deg_sh.at[pl.ds(s * sl, sl)])
        plsc.subcore_barrier()

        @pl.loop(0, chunks_per_tile)
        def _(j):
            pltpu.sync_copy(ones_v.at[pl.ds(0, _CB)],
                            deg_sh.at[idx_v.at[j]], add=True)

        plsc.subcore_barrier()

        @pl.when(c == 0)
        def _():
            pltpu.sync_copy(deg_sh.at[pl.ds(s * sl, sl)],
                            out.at[0].at[pl.ds(s * sl, sl)])

        @pl.when(c == 1)
        def _():
            pltpu.sync_copy(deg_sh.at[pl.ds(s * sl, sl)],
                            out.at[1].at[pl.ds(s * sl, sl)])

    return deg_kernel


def _make_sc_agg(n, dh, chunks_per_sub, nq):
    rows_per = n // _NS

    mesh = plsc.VectorSubcoreMesh(core_axis_name="c", subcore_axis_name="s")

    @functools.partial(
        pl.kernel,
        out_type=[jax.ShapeDtypeStruct((n, dh), jnp.float32)] * (2 * nq),
        mesh=mesh,
        scratch_types=[
            pltpu.VMEM((chunks_per_sub, _CB), jnp.int32),
            pltpu.VMEM((chunks_per_sub, _CB), jnp.int32),
        ] + [pltpu.VMEM((_CB, dh), jnp.float32)] * _NBUF
        + [pltpu.VMEM_SHARED((n, dh), jnp.float32)]
        + [pltpu.SemaphoreType.DMA] * (2 * _NBUF),
        compiler_params=pltpu.CompilerParams(use_tc_tiling_on_sc=False),
    )
    def agg_kernel(*refs):
        gs = refs[:2 * nq]
        src3, dst3 = refs[2 * nq], refs[2 * nq + 1]
        outs = refs[2 * nq + 2:4 * nq + 2]
        src_v, dst_v = refs[4 * nq + 2], refs[4 * nq + 3]
        rows = refs[4 * nq + 4:4 * nq + 4 + _NBUF]
        acc = refs[4 * nq + 4 + _NBUF]
        semg = refs[4 * nq + 5 + _NBUF:4 * nq + 5 + 2 * _NBUF]
        sems = refs[4 * nq + 5 + 2 * _NBUF:4 * nq + 5 + 3 * _NBUF]
        c = lax.axis_index("c")
        s = lax.axis_index("s")
        pltpu.sync_copy(src3.at[s], src_v)
        pltpu.sync_copy(dst3.at[s], dst_v)
        nch = chunks_per_sub
        assert nch % _NBUF == 0 and nch >= 2 * _NBUF

        def run(g, out):
            def startg(j, b):
                pltpu.async_copy(g.at[src_v.at[j]], rows[b], semg[b])

            def waitg(b):
                pltpu.make_async_copy(g.at[pl.ds(0, _CB)], rows[b],
                                      semg[b]).wait()

            def starts(j, b):
                pltpu.async_copy(rows[b], acc.at[dst_v.at[j]], sems[b],
                                 add=True)

            def waits(b):
                pltpu.make_async_copy(rows[b], acc.at[pl.ds(0, _CB)],
                                      sems[b]).wait()

            pltpu.sync_copy(g.at[pl.ds(s * rows_per, rows_per)],
                            acc.at[pl.ds(s * rows_per, rows_per)])
            plsc.subcore_barrier()

            for b in range(_NBUF):
                startg(b, b)

            @pl.loop(0, nch - _NBUF, step=_NBUF)
            def _(j):
                for b in range(_NBUF):
                    waitg(b)
                    starts(j + b, b)
                for b in range(_NBUF):
                    waits(b)
                    startg(j + _NBUF + b, b)

            for b in range(_NBUF):
                waitg(b)
                starts(nch - _NBUF + b, b)
            for b in range(_NBUF):
                waits(b)

            plsc.subcore_barrier()
            pltpu.sync_copy(acc.at[pl.ds(s * rows_per, rows_per)],
                            out.at[pl.ds(s * rows_per, rows_per)])

        @pl.when(c == 0)
        def _():
            for q in range(nq):
                run(gs[q], outs[q])

        @pl.when(c == 1)
        def _():
            for q in range(nq):
                run(gs[nq + q], outs[nq + q])

    return agg_kernel


def _dinv_block(dp):
    deg = dp[:, 0] + dp[:, 1] + 1.0
    return lax.rsqrt(jnp.maximum(deg, 1.0))


def _split_store(t, out_refs):
    w = t.shape[1] // len(out_refs)
    for q, ref in enumerate(out_refs):
        ref[...] = t[:, q * w:(q + 1) * w]


def _enc1_body(x_ref, dp_ref, w1_ref, *g_refs):
    dinv = _dinv_block(dp_ref[...])
    t = jnp.dot(x_ref[...], w1_ref[...], preferred_element_type=jnp.float32)
    _split_store(t * dinv[:, None], g_refs)


def _enc2_body(na, *refs):
    a_refs = refs[:na]
    dp_ref, w2_ref, b1_ref = refs[na:na + 3]
    g_refs = refs[na + 3:]
    dinv = _dinv_block(dp_ref[...])
    acc = jnp.concatenate([r[...] for r in a_refs], axis=1)
    h = jnp.maximum(acc * dinv[:, None] + b1_ref[...], 0.0)
    t = jnp.dot(h, w2_ref[...], preferred_element_type=jnp.float32)
    _split_store(t * dinv[:, None], g_refs)


def _dec_body(na, *refs):
    a_refs = refs[:na]
    (dp_ref, b2_ref, wd1_ref, bd1_ref, wd2_ref, bd2_ref,
     z_ref, xh_ref) = refs[na:]
    dinv = _dinv_block(dp_ref[...])
    acc = jnp.concatenate([r[...] for r in a_refs], axis=1)
    z = jnp.maximum(acc * dinv[:, None] + b2_ref[...], 0.0)
    z_ref[...] = z
    hh = jnp.maximum(
        jnp.dot(z, wd1_ref[...], preferred_element_type=jnp.float32)
        + bd1_ref[...], 0.0)
    xh_ref[...] = (jnp.dot(hh, wd2_ref[...], preferred_element_type=jnp.float32)
                   + bd2_ref[...])


def _adj_body(zi_ref, zj_ref, out_ref):
    out_ref[...] = lax.dot_general(
        zi_ref[...], zj_ref[...], (((1,), (1,)), ((), ())),
        preferred_element_type=jnp.float32)


def kernel(x, edge_index, W1, b1, W2, b2, Wd1, bd1, Wd2, bd2):
    n, in_dim = x.shape
    e = edge_index.shape[1]
    hid = W1.shape[1]
    lat = W2.shape[1]

    assert e % (_NS * _NC) == 0 and n % 8 == 0
    n_pad = ((n + 1024 - 1) // 1024) * 1024
    edges_per_tile = e // (_NC * _NS)
    deg_chunks = pl.cdiv(edges_per_tile, _CB)
    edges_per_sub = e // _NS
    agg_chunks = pl.cdiv(edges_per_sub, _CB)
    if agg_chunks % _NBUF:
        agg_chunks += _NBUF - agg_chunks % _NBUF

    src = edge_index[0]
    dst = edge_index[1]

    def _chunked(a, parts, nchunks, spread_fill):
        per = a.shape[0] // parts
        pad = nchunks * _CB - per
        if spread_fill:
            fill = (jnp.arange(pad, dtype=a.dtype) % (n_pad - n)) + n
        else:
            fill = jnp.zeros((pad,), a.dtype)
        filler = jnp.broadcast_to(fill, (parts, pad))
        return jnp.concatenate([a.reshape(parts, per), filler],
                               axis=1).reshape(parts, nchunks, _CB)

    dst3_deg = _chunked(dst, _NC * _NS, deg_chunks, True)
    src3 = _chunked(src, _NS, agg_chunks, False)
    dst3 = _chunked(dst, _NS, agg_chunks, True)
    ones_aux = jnp.ones((128,), jnp.float32)
    zeros_aux = jnp.zeros((n_pad,), jnp.float32)

    degp = _make_sc_degree(n_pad, deg_chunks)(dst3_deg, ones_aux, zeros_aux).T

    blk = n_pad // 16
    grid = 16
    row_spec = lambda d: pl.BlockSpec((blk, d), lambda i: (i, 0))
    dp_spec = pl.BlockSpec((blk, _NC), lambda i: (i, 0))
    full = lambda a: pl.BlockSpec(a.shape, lambda i: (0,) * a.ndim)

    b1r = b1.reshape(1, hid)
    b2r = b2.reshape(1, lat)
    bd1r = bd1.reshape(1, hid)
    bd2r = bd2.reshape(1, in_dim)

    nq1 = 2
    w1ch = hid // (2 * nq1)
    nq2 = 1
    w2ch = lat // (2 * nq2)

    g1s = pl.pallas_call(
        _enc1_body,
        grid=(grid,),
        in_specs=[row_spec(in_dim), dp_spec, full(W1)],
        out_specs=[row_spec(w1ch)] * (2 * nq1),
        out_shape=[jax.ShapeDtypeStruct((n_pad, w1ch), jnp.float32)]
        * (2 * nq1),
    )(x, degp, W1)

    a1s = _make_sc_agg(n_pad, w1ch, agg_chunks, nq1)(*g1s, src3, dst3)

    g2s = pl.pallas_call(
        functools.partial(_enc2_body, 2 * nq1),
        grid=(grid,),
        in_specs=[row_spec(w1ch)] * (2 * nq1) + [dp_spec, full(W2), full(b1r)],
        out_specs=[row_spec(w2ch)] * (2 * nq2),
        out_shape=[jax.ShapeDtypeStruct((n_pad, w2ch), jnp.float32)]
        * (2 * nq2),
    )(*a1s, degp, W2, b1r)

    a2s = _make_sc_agg(n_pad, w2ch, agg_chunks, nq2)(*g2s, src3, dst3)

    z, x_hat = pl.pallas_call(
        functools.partial(_dec_body, 2 * nq2),
        grid=(grid,),
        in_specs=[row_spec(w2ch)] * (2 * nq2)
        + [dp_spec, full(b2r), full(Wd1), full(bd1r), full(Wd2), full(bd2r)],
        out_specs=[row_spec(lat), row_spec(in_dim)],
        out_shape=[jax.ShapeDtypeStruct((n_pad, lat), jnp.float32),
                   jax.ShapeDtypeStruct((n_pad, in_dim), jnp.float32)],
    )(*a2s, degp, b2r, Wd1, bd1r, Wd2, bd2r)

    br, bc = 2560, 2560
    adj = pl.pallas_call(
        _adj_body,
        grid=(pl.cdiv(n, br), pl.cdiv(n, bc)),
        in_specs=[pl.BlockSpec((br, lat), lambda i, j: (i, 0)),
                  pl.BlockSpec((bc, lat), lambda i, j: (j, 0))],
        out_specs=pl.BlockSpec((br, bc), lambda i, j: (i, j)),
        out_shape=jax.ShapeDtypeStruct((n, n), jnp.float32),
    )(z, z)

    return (x_hat[:n], adj)


# speedup vs baseline: 1.1927x; 1.1927x over previous
import functools

import jax
import jax.numpy as jnp
from jax import lax
from jax.experimental import pallas as pl
from jax.experimental.pallas import tpu as pltpu
from jax.experimental.pallas import tpu_sc as plsc

_NC = 2
_NS = 16
_CB = 100
_NBUF = 4


def _make_sc_degree(n_pad, chunks_per_tile):
    sl = n_pad // _NS

    mesh = plsc.VectorSubcoreMesh(core_axis_name="c", subcore_axis_name="s")

    @functools.partial(
        pl.kernel,
        out_type=jax.ShapeDtypeStruct((_NC, n_pad), jnp.float32),
        mesh=mesh,
        scratch_types=[
            pltpu.VMEM((chunks_per_tile, _CB), jnp.int32),
            pltpu.VMEM((128,), jnp.float32),
            pltpu.VMEM_SHARED((n_pad,), jnp.float32),
        ],
    )
    def deg_kernel(dst3, ones_hbm, zeros_hbm, out, idx_v, ones_v, deg_sh):
        c = lax.axis_index("c")
        s = lax.axis_index("s")
        wid = c * _NS + s
        pltpu.sync_copy(dst3.at[wid], idx_v)
        pltpu.sync_copy(ones_hbm, ones_v)
        pltpu.sync_copy(zeros_hbm.at[pl.ds(s * sl, sl)],
                        deg_sh.at[pl.ds(s * sl, sl)])
        plsc.subcore_barrier()

        @pl.loop(0, chunks_per_tile)
        def _(j):
            pltpu.sync_copy(ones_v.at[pl.ds(0, _CB)],
                            deg_sh.at[idx_v.at[j]], add=True)

        plsc.subcore_barrier()

        @pl.when(c == 0)
        def _():
            pltpu.sync_copy(deg_sh.at[pl.ds(s * sl, sl)],
                            out.at[0].at[pl.ds(s * sl, sl)])

        @pl.when(c == 1)
        def _():
            pltpu.sync_copy(deg_sh.at[pl.ds(s * sl, sl)],
                            out.at[1].at[pl.ds(s * sl, sl)])

    return deg_kernel


def _make_sc_agg(n, dh, chunks_per_sub, nq):
    rows_per = n // _NS

    mesh = plsc.VectorSubcoreMesh(core_axis_name="c", subcore_axis_name="s")

    @functools.partial(
        pl.kernel,
        out_type=[jax.ShapeDtypeStruct((n, dh), jnp.float32)] * (2 * nq),
        mesh=mesh,
        scratch_types=[
            pltpu.VMEM((chunks_per_sub, _CB), jnp.int32),
            pltpu.VMEM((chunks_per_sub, _CB), jnp.int32),
        ] + [pltpu.VMEM((_CB, dh), jnp.float32)] * _NBUF
        + [pltpu.VMEM_SHARED((n, dh), jnp.float32)]
        + [pltpu.SemaphoreType.DMA] * (2 * _NBUF),
        compiler_params=pltpu.CompilerParams(use_tc_tiling_on_sc=False),
    )
    def agg_kernel(*refs):
        gs = refs[:2 * nq]
        src3, dst3 = refs[2 * nq], refs[2 * nq + 1]
        outs = refs[2 * nq + 2:4 * nq + 2]
        src_v, dst_v = refs[4 * nq + 2], refs[4 * nq + 3]
        rows = refs[4 * nq + 4:4 * nq + 4 + _NBUF]
        acc = refs[4 * nq + 4 + _NBUF]
        semg = refs[4 * nq + 5 + _NBUF:4 * nq + 5 + 2 * _NBUF]
        sems = refs[4 * nq + 5 + 2 * _NBUF:4 * nq + 5 + 3 * _NBUF]
        c = lax.axis_index("c")
        s = lax.axis_index("s")
        pltpu.sync_copy(src3.at[s], src_v)
        pltpu.sync_copy(dst3.at[s], dst_v)
        nch = chunks_per_sub
        assert nch % _NBUF == 0 and nch >= 2 * _NBUF

        def run(g, out):
            def startg(j, b):
                pltpu.async_copy(g.at[src_v.at[j]], rows[b], semg[b])

            def waitg(b):
                pltpu.make_async_copy(g.at[pl.ds(0, _CB)], rows[b],
                                      semg[b]).wait()

            def starts(j, b):
                pltpu.async_copy(rows[b], acc.at[dst_v.at[j]], sems[b],
                                 add=True)

            def waits(b):
                pltpu.make_async_copy(rows[b], acc.at[pl.ds(0, _CB)],
                                      sems[b]).wait()

            pltpu.sync_copy(g.at[pl.ds(s * rows_per, rows_per)],
                            acc.at[pl.ds(s * rows_per, rows_per)])
            plsc.subcore_barrier()

            for b in range(_NBUF):
                startg(b, b)

            @pl.loop(0, nch - _NBUF, step=_NBUF)
            def _(j):
                for b in range(_NBUF):
                    waitg(b)
                    starts(j + b, b)
                for b in range(_NBUF):
                    waits(b)
                    startg(j + _NBUF + b, b)

            for b in range(_NBUF):
                waitg(b)
                starts(nch - _NBUF + b, b)
            for b in range(_NBUF):
                waits(b)

            plsc.subcore_barrier()
            pltpu.sync_copy(acc.at[pl.ds(s * rows_per, rows_per)],
                            out.at[pl.ds(s * rows_per, rows_per)])

        @pl.when(c == 0)
        def _():
            for q in range(nq):
                run(gs[q], outs[q])

        @pl.when(c == 1)
        def _():
            for q in range(nq):
                run(gs[nq + q], outs[nq + q])

    return agg_kernel


def _dinv_block(dp):
    deg = dp[:, 0] + dp[:, 1] + 1.0
    return lax.rsqrt(jnp.maximum(deg, 1.0))


def _split_store(t, out_refs):
    w = t.shape[1] // len(out_refs)
    for q, ref in enumerate(out_refs):
        ref[...] = t[:, q * w:(q + 1) * w]


def _enc1_body(x_ref, dp_ref, w1_ref, *g_refs):
    dinv = _dinv_block(dp_ref[...])
    t = jnp.dot(x_ref[...], w1_ref[...], preferred_element_type=jnp.float32)
    _split_store(t * dinv[:, None], g_refs)


def _enc2_body(na, *refs):
    a_refs = refs[:na]
    dp_ref, w2_ref, b1_ref = refs[na:na + 3]
    g_refs = refs[na + 3:]
    dinv = _dinv_block(dp_ref[...])
    acc = jnp.concatenate([r[...] for r in a_refs], axis=1)
    h = jnp.maximum(acc * dinv[:, None] + b1_ref[...], 0.0)
    t = jnp.dot(h, w2_ref[...], preferred_element_type=jnp.float32)
    _split_store(t * dinv[:, None], g_refs)


def _dec_body(na, *refs):
    a_refs = refs[:na]
    (dp_ref, b2_ref, wd1_ref, bd1_ref, wd2_ref, bd2_ref,
     z_ref, xh_ref) = refs[na:]
    dinv = _dinv_block(dp_ref[...])
    acc = jnp.concatenate([r[...] for r in a_refs], axis=1)
    z = jnp.maximum(acc * dinv[:, None] + b2_ref[...], 0.0)
    z_ref[...] = z
    hh = jnp.maximum(
        jnp.dot(z, wd1_ref[...], preferred_element_type=jnp.float32)
        + bd1_ref[...], 0.0)
    xh_ref[...] = (jnp.dot(hh, wd2_ref[...], preferred_element_type=jnp.float32)
                   + bd2_ref[...])


def _adj_body(zi_ref, zj_ref, out_ref):
    out_ref[...] = lax.dot_general(
        zi_ref[...], zj_ref[...], (((1,), (1,)), ((), ())),
        preferred_element_type=jnp.float32)


def kernel(x, edge_index, W1, b1, W2, b2, Wd1, bd1, Wd2, bd2):
    n, in_dim = x.shape
    e = edge_index.shape[1]
    hid = W1.shape[1]
    lat = W2.shape[1]

    assert e % (_NS * _NC) == 0 and n % 8 == 0
    n_pad = ((n + 1024 - 1) // 1024) * 1024
    edges_per_tile = e // (_NC * _NS)
    deg_chunks = pl.cdiv(edges_per_tile, _CB)
    edges_per_sub = e // _NS
    agg_chunks = pl.cdiv(edges_per_sub, _CB)
    if agg_chunks % _NBUF:
        agg_chunks += _NBUF - agg_chunks % _NBUF

    src = edge_index[0]
    dst = edge_index[1]

    def _chunked(a, parts, nchunks, spread_fill):
        per = a.shape[0] // parts
        pad = nchunks * _CB - per
        if spread_fill:
            fill = (jnp.arange(pad, dtype=a.dtype) % (n_pad - n)) + n
        else:
            fill = jnp.zeros((pad,), a.dtype)
        filler = jnp.broadcast_to(fill, (parts, pad))
        return jnp.concatenate([a.reshape(parts, per), filler],
                               axis=1).reshape(parts, nchunks, _CB)

    dst3_deg = _chunked(dst, _NC * _NS, deg_chunks, True)
    src3 = _chunked(src, _NS, agg_chunks, False)
    dst3 = _chunked(dst, _NS, agg_chunks, True)
    ones_aux = jnp.ones((128,), jnp.float32)
    zeros_aux = jnp.zeros((n_pad,), jnp.float32)

    degp = _make_sc_degree(n_pad, deg_chunks)(dst3_deg, ones_aux, zeros_aux).T

    blk = n_pad // 16
    grid = 16
    row_spec = lambda d: pl.BlockSpec((blk, d), lambda i: (i, 0))
    dp_spec = pl.BlockSpec((blk, _NC), lambda i: (i, 0))
    full = lambda a: pl.BlockSpec(a.shape, lambda i: (0,) * a.ndim)

    b1r = b1.reshape(1, hid)
    b2r = b2.reshape(1, lat)
    bd1r = bd1.reshape(1, hid)
    bd2r = bd2.reshape(1, in_dim)

    nq1 = 2
    w1ch = hid // (2 * nq1)
    nq2 = 1
    w2ch = lat // (2 * nq2)

    g1s = pl.pallas_call(
        _enc1_body,
        grid=(grid,),
        in_specs=[row_spec(in_dim), dp_spec, full(W1)],
        out_specs=[row_spec(w1ch)] * (2 * nq1),
        out_shape=[jax.ShapeDtypeStruct((n_pad, w1ch), jnp.float32)]
        * (2 * nq1),
    )(x, degp, W1)

    a1s = _make_sc_agg(n_pad, w1ch, agg_chunks, nq1)(*g1s, src3, dst3)

    g2s = pl.pallas_call(
        functools.partial(_enc2_body, 2 * nq1),
        grid=(grid,),
        in_specs=[row_spec(w1ch)] * (2 * nq1) + [dp_spec, full(W2), full(b1r)],
        out_specs=[row_spec(w2ch)] * (2 * nq2),
        out_shape=[jax.ShapeDtypeStruct((n_pad, w2ch), jnp.float32)]
        * (2 * nq2),
    )(*a1s, degp, W2, b1r)

    a2s = _make_sc_agg(n_pad, w2ch, agg_chunks, nq2)(*g2s, src3, dst3)

    z, x_hat = pl.pallas_call(
        functools.partial(_dec_body, 2 * nq2),
        grid=(grid,),
        in_specs=[row_spec(w2ch)] * (2 * nq2)
        + [dp_spec, full(b2r), full(Wd1), full(bd1r), full(Wd2), full(bd2r)],
        out_specs=[row_spec(lat), row_spec(in_dim)],
        out_shape=[jax.ShapeDtypeStruct((n_pad, lat), jnp.float32),
                   jax.ShapeDtypeStruct((n_pad, in_dim), jnp.float32)],
    )(*a2s, degp, b2r, Wd1, bd1r, Wd2, bd2r)

    br, bc = 2560, 2560
    adj = pl.pallas_call(
        _adj_body,
        grid=(pl.cdiv(n, br), pl.cdiv(n, bc)),
        in_specs=[pl.BlockSpec((br, lat), lambda i, j: (i, 0)),
                  pl.BlockSpec((bc, lat), lambda i, j: (j, 0))],
        out_specs=pl.BlockSpec((br, bc), lambda i, j: (i, j)),
        out_shape=jax.ShapeDtypeStruct((n, n), jnp.float32),
    )(z, z)

    return (x_hat[:n], adj)
